# baseline (speedup 1.0000x reference)
import functools

import jax
import jax.numpy as jnp
import numpy as np
from jax import lax
from jax.experimental import pallas as pl
from jax.experimental.pallas import tpu as pltpu
from jax.experimental.pallas import tpu_sc as plsc

VOCAB = 100000
D = 128
B = 4096
S = 200
HALF_S = S // 2
DI = D // 2
NC, NS = 2, 16
NW = NC * NS
B_PER_W = B // NW
NLANE = 16
NSLOT = D // NLANE


def _pool_body(ids_hbm, table_hbm, out_hbm, idx_v, buf_v, stage_v, sem0, sem1):
  wid = lax.axis_index("s") * NC + lax.axis_index("c")
  base = wid * B_PER_W

  pltpu.sync_copy(ids_hbm.at[pl.ds(2 * base, 2 * B_PER_W)], idx_v)

  def fire(row, slot, sem):
    pltpu.async_copy(table_hbm.at[idx_v.at[2 * row]],
                     buf_v.at[slot, pl.ds(0, HALF_S)], sem)
    pltpu.async_copy(table_hbm.at[idx_v.at[2 * row + 1]],
                     buf_v.at[slot, pl.ds(HALF_S, HALF_S)], sem)

  def drain(slot, sem):
    pltpu.make_async_copy(table_hbm.at[pl.ds(0, S)], buf_v.at[slot], sem).wait()

  def accumulate(row, slot):
    def body(s, accs):
      accs = list(accs)
      for k in range(NSLOT // 2):
        w = buf_v[slot, s, pl.ds(k * NLANE, NLANE)]
        a = lax.bitcast_convert_type(lax.shift_left(w, 16), jnp.float32)
        b = lax.bitcast_convert_type(lax.bitwise_and(w, jnp.int32(-65536)),
                                     jnp.float32)
        accs[2 * k] = accs[2 * k] + a
        accs[2 * k + 1] = accs[2 * k + 1] + b
      return tuple(accs)
    accs = lax.fori_loop(
        0, S, body, tuple(jnp.zeros((NLANE,), jnp.float32)
                          for _ in range(NSLOT)))
    for k in range(NSLOT):
      stage_v[row, pl.ds(k * NLANE, NLANE)] = accs[k]

  fire(0, 0, sem0)
  fire(1, 1, sem1)

  def outer(i, _):
    row0 = 2 * i
    drain(0, sem0)
    @pl.when(row0 + 2 < B_PER_W)
    def _():
      fire(row0 + 2, 0, sem0)
    accumulate(row0, 0)
    drain(1, sem1)
    @pl.when(row0 + 3 < B_PER_W)
    def _():
      fire(row0 + 3, 1, sem1)
    accumulate(row0 + 1, 1)
    return 0

  lax.fori_loop(0, B_PER_W // 2, outer, 0)
  pltpu.sync_copy(stage_v, out_hbm.at[pl.ds(base, B_PER_W)])


def _pool(ids2, table):
  mesh = plsc.VectorSubcoreMesh(core_axis_name="c", subcore_axis_name="s",
                                num_cores=NC, num_subcores=NS)
  f = pl.kernel(
      _pool_body,
      out_type=jax.ShapeDtypeStruct((B, D), jnp.float32),
      mesh=mesh,
      scratch_types=[
          pltpu.VMEM((2 * B_PER_W, HALF_S), jnp.int32),
          pltpu.VMEM((2, S, DI), jnp.int32),
          pltpu.VMEM((B_PER_W, D), jnp.float32),
          pltpu.SemaphoreType.DMA,
          pltpu.SemaphoreType.DMA,
      ],
      compiler_params=pltpu.CompilerParams(use_tc_tiling_on_sc=False),
  )
  return f(ids2, table)


def _mlp_body(pooled_ref, mask_ref, w1_ref, b1_ref, w2_ref, b2_ref, out_ref):
  denom = jnp.sum(mask_ref[...], axis=1, keepdims=True)
  pooled = pooled_ref[...] / denom
  h = lax.dot_general(pooled, w1_ref[...], (((1,), (1,)), ((), ())),
                      preferred_element_type=jnp.float32)
  h = jnp.maximum(h + b1_ref[...], 0.0)
  out = lax.dot_general(h, w2_ref[...], (((1,), (1,)), ((), ())),
                        preferred_element_type=jnp.float32)
  out_ref[...] = out + b2_ref[0]


def _mlp(pooled_sum, mask, W1, b1, W2, b2):
  blk = 1024
  grid = (B // blk,)
  return pl.pallas_call(
      _mlp_body,
      grid=grid,
      in_specs=[
          pl.BlockSpec((blk, D), lambda i: (i, 0)),
          pl.BlockSpec((blk, S), lambda i: (i, 0)),
          pl.BlockSpec((D, D), lambda i: (0, 0)),
          pl.BlockSpec((1, D), lambda i: (0, 0)),
          pl.BlockSpec((8, D), lambda i: (0, 0)),
          pl.BlockSpec(memory_space=pltpu.SMEM),
      ],
      out_specs=pl.BlockSpec((blk, 8), lambda i: (i, 0)),
      out_shape=jax.ShapeDtypeStruct((B, 8), jnp.float32),
  )(pooled_sum, mask, W1, b1.reshape(1, D), jnp.pad(W2, ((0, 7), (0, 0))),
    b2)[:, 0:1]


_PERM = np.arange(D).reshape(D // 32, 16, 2).transpose(0, 2, 1).reshape(D)


def kernel(input_ids, attention_mask, emb_table, W1, b1, W2, b2):
  ids2 = input_ids.reshape(2 * B, HALF_S)
  table_i32 = lax.bitcast_convert_type(
      emb_table.astype(jnp.bfloat16).reshape(VOCAB, DI, 2), jnp.int32)
  pooled_sum = _pool(ids2, table_i32)
  return _mlp(pooled_sum, attention_mask, W1[:, _PERM], b1, W2, b2)


# speedup vs baseline: 3.5348x; 3.5348x over previous
import jax
import jax.numpy as jnp
from jax import lax
from jax.experimental import pallas as pl
from jax.experimental.pallas import tpu as pltpu
from jax.experimental.pallas import tpu_sc as plsc

VOCAB = 100000
D = 128
B = 4096
S = 200
HALF_S = S // 2
NC, NS = 2, 16
NW = NC * NS
B_PER_W = B // NW
C_PER_W = 2 * B_PER_W
NSLOTS = 4
NLANE = 16
NREG = D // NLANE


def _pool_body(ids_hbm, table_hbm, dummy_hbm, out_hbm, idx_v, buf_v, stage_v,
               *sems):
  wid = lax.axis_index("s") * NC + lax.axis_index("c")
  base = wid * B_PER_W

  pltpu.sync_copy(ids_hbm.at[pl.ds(2 * base, C_PER_W)], idx_v)

  def fire(chunk, slot):
    pltpu.async_copy(table_hbm.at[idx_v.at[chunk]], buf_v.at[slot],
                     sems[slot])

  def drain(slot):
    pltpu.make_async_copy(dummy_hbm, buf_v.at[slot], sems[slot]).wait()

  def accumulate(slot, accs):
    def body(s, accs):
      return tuple(accs[k] + buf_v[slot, s, pl.ds(k * NLANE, NLANE)]
                   for k in range(NREG))
    return lax.fori_loop(0, HALF_S, body, accs)

  zeros = tuple(jnp.zeros((NLANE,), jnp.float32) for _ in range(NREG))

  for slot in range(NSLOTS):
    fire(slot, slot)

  def outer(i, accs):
    c0 = NSLOTS * i
    for j in range(NSLOTS):
      slot = j
      c = c0 + j
      drain(slot)
      @pl.when(c + NSLOTS < C_PER_W)
      def _():
        fire(c + NSLOTS, slot)
      accs = accumulate(slot, accs)
      if j % 2 == 1:
        row = (c0 + j) // 2
        for k in range(NREG):
          stage_v[row, pl.ds(k * NLANE, NLANE)] = accs[k]
        accs = zeros
    return accs

  lax.fori_loop(0, C_PER_W // NSLOTS, outer, zeros)
  pltpu.sync_copy(stage_v, out_hbm.at[pl.ds(base, B_PER_W)])


def _pool(ids2, table):
  mesh = plsc.VectorSubcoreMesh(core_axis_name="c", subcore_axis_name="s",
                                num_cores=NC, num_subcores=NS)
  f = pl.kernel(
      _pool_body,
      out_type=jax.ShapeDtypeStruct((B, D), jnp.float32),
      mesh=mesh,
      scratch_types=[
          pltpu.VMEM((C_PER_W, HALF_S), jnp.int32),
          pltpu.VMEM((NSLOTS, HALF_S, D), jnp.float32),
          pltpu.VMEM((B_PER_W, D), jnp.float32),
      ] + [pltpu.SemaphoreType.DMA] * NSLOTS,
  )
  return f(ids2, table, jnp.zeros((HALF_S, D), jnp.float32))


def _mlp_body(pooled_ref, mask_ref, w1_ref, b1_ref, w2_ref, b2_ref, out_ref):
  denom = jnp.sum(mask_ref[...], axis=1, keepdims=True)
  pooled = pooled_ref[...] / denom
  h = lax.dot_general(pooled, w1_ref[...], (((1,), (1,)), ((), ())),
                      preferred_element_type=jnp.float32)
  h = jnp.maximum(h + b1_ref[...], 0.0)
  out = lax.dot_general(h, w2_ref[...], (((1,), (1,)), ((), ())),
                        preferred_element_type=jnp.float32)
  out_ref[...] = out + b2_ref[0]


def _mlp(pooled_sum, mask, W1, b1, W2, b2):
  blk = 2048
  grid = (B // blk,)
  return pl.pallas_call(
      _mlp_body,
      grid=grid,
      in_specs=[
          pl.BlockSpec((blk, D), lambda i: (i, 0)),
          pl.BlockSpec((blk, S), lambda i: (i, 0)),
          pl.BlockSpec((D, D), lambda i: (0, 0)),
          pl.BlockSpec((1, D), lambda i: (0, 0)),
          pl.BlockSpec((8, D), lambda i: (0, 0)),
          pl.BlockSpec(memory_space=pltpu.SMEM),
      ],
      out_specs=pl.BlockSpec((blk, 8), lambda i: (i, 0)),
      out_shape=jax.ShapeDtypeStruct((B, 8), jnp.float32),
  )(pooled_sum, mask, W1, b1.reshape(1, D), jnp.pad(W2, ((0, 7), (0, 0))),
    b2)[:, 0:1]


def kernel(input_ids, attention_mask, emb_table, W1, b1, W2, b2):
  ids2 = input_ids.reshape(2 * B, HALF_S)
  pooled_sum = _pool(ids2, emb_table)
  return _mlp(pooled_sum, attention_mask, W1, b1, W2, b2)
